# Initial kernel scaffold; baseline (speedup 1.0000x reference)
import jax
import jax.numpy as jnp
from jax.experimental import pallas as pl


def kernel(x_t, c_0_pred, mask_generate, t, alpha_bars):
    raise NotImplementedError("write your pallas kernel here")


# speedup vs baseline: 1.7161x; 1.7161x over previous
import functools

import jax
import jax.numpy as jnp
from jax.experimental import pallas as pl

_NUM_STEPS = 100
_K = 20
_N = 131072
_BLK = 2048


def _fused_kernel(xt_ref, t_ref, mask_ref, ab_ref, c0_ref, g_ref,
                  post_ref, xnext_ref):
    K = _K
    lane_k = jax.lax.broadcasted_iota(jnp.int32, (1, K), 1)

    x_t = xt_ref[...]
    c_t = (lane_k == x_t).astype(jnp.float32)

    ab = ab_ref[...]
    lane128 = jax.lax.broadcasted_iota(jnp.int32, (1, 128), 1)
    t_val = t_ref[...]
    onehot_t = (lane128 == t_val).astype(jnp.float32)
    alpha = jnp.sum(onehot_t * ab, axis=1, keepdims=True)

    c0 = c0_ref[...]
    u = (1.0 - alpha) * (1.0 / K)
    theta = (alpha * c_t + u) * (alpha * c0 + u)
    s = jnp.sum(theta, axis=1, keepdims=True)
    theta = theta / (s + 1e-8)

    m = mask_ref[...]
    post = theta * m + c_t * (1.0 - m)
    post_ref[...] = post

    scores = jnp.log(post + 1e-12) + g_ref[...]
    mx = jnp.max(scores, axis=1, keepdims=True)
    idx = jnp.min(jnp.where(scores == mx, lane_k, K), axis=1, keepdims=True)
    xnext_ref[...] = idx


@functools.partial(jax.jit, static_argnames=())
def kernel(x_t, c_0_pred, mask_generate, t, alpha_bars):
    N, K = c_0_pred.shape
    g = jax.random.gumbel(jax.random.key(1), (N, K), jnp.float32)
    xt2 = x_t.astype(jnp.int32).reshape(N, 1)
    t2 = t.astype(jnp.int32).reshape(N, 1)
    m2 = mask_generate.astype(jnp.float32).reshape(N, 1)
    ab = jnp.zeros((1, 128), jnp.float32).at[0, : alpha_bars.shape[0]].set(alpha_bars)

    nblk = N // _BLK
    post, xnext = pl.pallas_call(
        _fused_kernel,
        grid=(nblk,),
        in_specs=[
            pl.BlockSpec((_BLK, 1), lambda i: (i, 0)),
            pl.BlockSpec((_BLK, 1), lambda i: (i, 0)),
            pl.BlockSpec((_BLK, 1), lambda i: (i, 0)),
            pl.BlockSpec((1, 128), lambda i: (0, 0)),
            pl.BlockSpec((_BLK, K), lambda i: (i, 0)),
            pl.BlockSpec((_BLK, K), lambda i: (i, 0)),
        ],
        out_specs=[
            pl.BlockSpec((_BLK, K), lambda i: (i, 0)),
            pl.BlockSpec((_BLK, 1), lambda i: (i, 0)),
        ],
        out_shape=[
            jax.ShapeDtypeStruct((N, K), jnp.float32),
            jax.ShapeDtypeStruct((N, 1), jnp.int32),
        ],
    )(xt2, t2, m2, ab, c_0_pred, g)
    return (post, xnext)
